# baseline (speedup 1.0000x reference)
import functools

import jax
import jax.numpy as jnp
from jax import lax
from jax.experimental import pallas as pl
from jax.experimental.pallas import tpu as pltpu
from jax.experimental.pallas import tpu_sc as plsc

_VOCAB = 1_000_000
_MAXLEN = 200
_EMBED = 32
_BATCH = 4096

_L = 16
_NC = 2
_NS = 16
_NW = _NC * _NS
_BB = _BATCH // _NW
_TB = 8
_NTB = _MAXLEN // _TB
_PAIRS = (_NTB - 1) // 2

_mesh = plsc.VectorSubcoreMesh(core_axis_name="c", subcore_axis_name="s")


@functools.partial(
    pl.kernel,
    out_type=jax.ShapeDtypeStruct((_MAXLEN, _EMBED // 8, _NW, 8, 128), jnp.float32),
    mesh=_mesh,
    scratch_types=[
        pltpu.VMEM((_BB, _TB), jnp.int32),
        pltpu.VMEM((_BB, _TB), jnp.int32),
        pltpu.VMEM((_TB, _BB), jnp.int32),
        pltpu.VMEM((_TB, _BB), jnp.int32),
        pltpu.VMEM((_TB * _BB, _EMBED), jnp.float32),
        pltpu.VMEM((_TB * _BB, _EMBED), jnp.float32),
        pltpu.VMEM((_TB, _EMBED // 8, 1, 8, _BB), jnp.float32),
        pltpu.VMEM((_MAXLEN, _EMBED), jnp.float32),
        pltpu.SemaphoreType.DMA,
        pltpu.SemaphoreType.DMA,
    ],
    compiler_params=pltpu.CompilerParams(use_tc_tiling_on_sc=False,
                                         needs_layout_passes=False),
)
def _tok_pos_embed(x_hbm, tok_hbm, pos_hbm, out_hbm,
                   xblk0, xblk1, idx0, idx1, rows0, rows1,
                   obuf_v, pos_v, sem0, sem1):
    w = lax.axis_index("s") * _NC + lax.axis_index("c")
    pltpu.sync_copy(pos_hbm, pos_v)

    lane = jnp.arange(_L, dtype=jnp.int32)
    et_lo = lane >> 3
    es_lo = lane & 7
    et_hi = (lane + _L) >> 3
    es_hi = (lane + _L) & 7
    zero = jnp.zeros((_L,), dtype=jnp.int32)

    xblks = (xblk0, xblk1)
    idxs = (idx0, idx1)
    rows = (rows0, rows1)
    sems = (sem0, sem1)

    def stage(buf, tt):
        xblk, idx_v = xblks[buf], idxs[buf]
        pltpu.sync_copy(
            x_hbm.at[pl.ds(w * _BB, _BB), pl.ds(tt * _TB, _TB)], xblk)
        for s in range(_TB):
            scol = jnp.full((_L,), s, dtype=jnp.int32)
            for lg in range(_BB // _L):
                v = plsc.load_gather(xblk, [lane + (lg * _L), scol])
                idx_v[s, pl.ds(lg * _L, _L)] = v

    def fire(buf):
        for s in range(_TB):
            pltpu.make_async_copy(
                tok_hbm.at[idxs[buf].at[s]],
                rows[buf].at[pl.ds(s * _BB, _BB)],
                sems[buf],
            ).start()

    def drain(buf):
        for s in range(_TB):
            pltpu.make_async_copy(
                tok_hbm.at[idxs[buf].at[s]],
                rows[buf].at[pl.ds(s * _BB, _BB)],
                sems[buf],
            ).wait()

    def scatter_out(buf, tt):
        rows_v = rows[buf]
        lane128 = lane << 7
        for s in range(_TB):
            t = tt * _TB + s
            pos_lo = pos_v[t, pl.ds(0, _L)]
            pos_hi = pos_v[t, pl.ds(_L, _L)]

            def lblock(lb, acc):
                j0 = s * _BB + lb * _L
                jsplat = jnp.full((_L,), j0, dtype=jnp.int32)
                dbase = lane128 + (s * 4096 + lb * _L)

                def diag(d, acc2):
                    rot = (lane + d) & 15
                    jvec = jsplat + rot
                    v0 = plsc.load_gather(rows_v, [jvec, lane]) + pos_lo
                    v1 = plsc.load_gather(rows_v, [jvec, lane + _L]) + pos_hi
                    off = dbase + rot
                    plsc.store_scatter(obuf_v, [zero, zero, zero, zero, off], v0)
                    plsc.store_scatter(obuf_v, [zero, zero, zero, zero, off + 2048], v1)
                    return acc2

                lax.fori_loop(0, _L, diag, 0, unroll=4)
                return acc

            lax.fori_loop(0, _BB // _L, lblock, 0)

        pltpu.sync_copy(
            obuf_v,
            out_hbm.at[pl.ds(tt * _TB, _TB), slice(None), pl.ds(w, 1)])

    stage(0, 0)
    fire(0)

    def double_step(tt2, carry):
        tt_e = tt2 * 2
        stage(1, tt_e + 1)
        fire(1)
        drain(0)
        scatter_out(0, tt_e)
        stage(0, tt_e + 2)
        fire(0)
        drain(1)
        scatter_out(1, tt_e + 1)
        return carry

    lax.fori_loop(0, _PAIRS, double_step, 0)

    drain(0)
    scatter_out(0, _NTB - 1)


_CB = 800
_BLKS = _VOCAB // _CB


@functools.partial(
    pl.kernel,
    out_type=jax.ShapeDtypeStruct((_VOCAB, _EMBED), jnp.float32),
    mesh=_mesh,
    scratch_types=[
        pltpu.VMEM((_EMBED, _CB + 1), jnp.float32),
        pltpu.VMEM((_CB, _EMBED), jnp.float32),
        pltpu.SemaphoreType.DMA,
    ],
    compiler_params=pltpu.CompilerParams(use_tc_tiling_on_sc=False,
                                         needs_layout_passes=False),
)
def _table_transpose(tokT_hbm, out_hbm, inb_v, outb_v, sem):
    w = lax.axis_index("s") * _NC + lax.axis_index("c")
    nblk = jnp.where(w < 2, 40, 39)
    blk0 = w * 39 + jnp.minimum(w, 2)

    lane = jnp.arange(_L, dtype=jnp.int32)
    adr_lo = lane * (_CB + 1)
    adr_hi = (lane + _L) * (_CB + 1)

    def block(i, carry):
        c0 = (blk0 + i) * _CB
        pltpu.sync_copy(tokT_hbm.at[slice(None), pl.ds(c0, _CB)],
                        inb_v.at[slice(None), pl.ds(0, _CB)])

        def col(c, carry2):
            csplat = jnp.full((_L,), c, dtype=jnp.int32)
            outb_v[c, pl.ds(0, _L)] = plsc.load_gather(inb_v, [zerocol, adr_lo + csplat])
            outb_v[c, pl.ds(_L, _L)] = plsc.load_gather(inb_v, [zerocol, adr_hi + csplat])
            return carry2

        zerocol = jnp.zeros((_L,), dtype=jnp.int32)
        lax.fori_loop(0, _CB, col, 0, unroll=8)

        pltpu.sync_copy(outb_v, out_hbm.at[pl.ds(c0, _CB)])
        return carry

    lax.fori_loop(0, nblk, block, 0)


def kernel(x, token_table, pos_table):
    table_rm = _table_transpose(token_table.T)
    out5 = _tok_pos_embed(x.astype(jnp.int32), table_rm, pos_table)
    return out5.transpose(2, 4, 0, 1, 3).reshape(_BATCH, _MAXLEN, _EMBED)


# speedup vs baseline: 5.9178x; 5.9178x over previous
import functools

import jax
import jax.numpy as jnp
from jax import lax
from jax.experimental import pallas as pl
from jax.experimental.pallas import tpu as pltpu
from jax.experimental.pallas import tpu_sc as plsc

_VOCAB = 1_000_000
_MAXLEN = 200
_EMBED = 32
_BATCH = 4096

_L = 16
_NC = 2
_NS = 16
_NW = _NC * _NS
_BB = _BATCH // _NW
_TB = 8
_NTB = _MAXLEN // _TB
_PAIRS = (_NTB - 1) // 2

_mesh = plsc.VectorSubcoreMesh(core_axis_name="c", subcore_axis_name="s")


@functools.partial(
    pl.kernel,
    out_type=jax.ShapeDtypeStruct((_MAXLEN, _EMBED // 8, _NW, 8, 128), jnp.float32),
    mesh=_mesh,
    scratch_types=[
        pltpu.VMEM((_BB, _TB), jnp.int32),
        pltpu.VMEM((_BB, _TB), jnp.int32),
        pltpu.VMEM((_TB, _BB), jnp.int32),
        pltpu.VMEM((_TB, _BB), jnp.int32),
        pltpu.VMEM((_TB * _BB, _EMBED), jnp.float32),
        pltpu.VMEM((_TB * _BB, _EMBED), jnp.float32),
        pltpu.VMEM((_TB, _EMBED // 8, 1, 8, _BB), jnp.float32),
        pltpu.VMEM((_MAXLEN, _EMBED), jnp.float32),
        pltpu.SemaphoreType.DMA,
        pltpu.SemaphoreType.DMA,
    ],
    compiler_params=pltpu.CompilerParams(use_tc_tiling_on_sc=False,
                                         needs_layout_passes=False),
)
def _tok_pos_embed(x_hbm, tok_hbm, pos_hbm, out_hbm,
                   xblk0, xblk1, idx0, idx1, rows0, rows1,
                   obuf_v, pos_v, sem0, sem1):
    w = lax.axis_index("s") * _NC + lax.axis_index("c")
    pltpu.sync_copy(pos_hbm, pos_v)

    lane = jnp.arange(_L, dtype=jnp.int32)
    et_lo = lane >> 3
    es_lo = lane & 7
    et_hi = (lane + _L) >> 3
    es_hi = (lane + _L) & 7
    zero = jnp.zeros((_L,), dtype=jnp.int32)

    xblks = (xblk0, xblk1)
    idxs = (idx0, idx1)
    rows = (rows0, rows1)
    sems = (sem0, sem1)

    def stage(buf, tt):
        xblk, idx_v = xblks[buf], idxs[buf]
        pltpu.sync_copy(
            x_hbm.at[pl.ds(w * _BB, _BB), pl.ds(tt * _TB, _TB)], xblk)
        for s in range(_TB):
            scol = jnp.full((_L,), s, dtype=jnp.int32)
            for lg in range(_BB // _L):
                v = plsc.load_gather(xblk, [lane + (lg * _L), scol])
                idx_v[s, pl.ds(lg * _L, _L)] = v

    def fire(buf):
        for s in range(_TB):
            pltpu.make_async_copy(
                tok_hbm.at[idxs[buf].at[s]],
                rows[buf].at[pl.ds(s * _BB, _BB)],
                sems[buf],
            ).start()

    def drain(buf):
        for s in range(_TB):
            pltpu.make_async_copy(
                tok_hbm.at[idxs[buf].at[s]],
                rows[buf].at[pl.ds(s * _BB, _BB)],
                sems[buf],
            ).wait()

    def scatter_out(buf, tt):
        rows_v = rows[buf]
        lane128 = lane << 7
        for s in range(_TB):
            t = tt * _TB + s
            pos_lo = pos_v[t, pl.ds(0, _L)]
            pos_hi = pos_v[t, pl.ds(_L, _L)]

            def lblock(lb, acc):
                j0 = s * _BB + lb * _L
                jsplat = jnp.full((_L,), j0, dtype=jnp.int32)
                dbase = lane128 + (s * 4096 + lb * _L)

                def diag(d, acc2):
                    rot = (lane + d) & 15
                    jvec = jsplat + rot
                    v0 = plsc.load_gather(rows_v, [jvec, lane]) + pos_lo
                    v1 = plsc.load_gather(rows_v, [jvec, lane + _L]) + pos_hi
                    off = dbase + rot
                    plsc.store_scatter(obuf_v, [zero, zero, zero, zero, off], v0)
                    plsc.store_scatter(obuf_v, [zero, zero, zero, zero, off + 2048], v1)
                    return acc2

                lax.fori_loop(0, _L, diag, 0, unroll=4)
                return acc

            lax.fori_loop(0, _BB // _L, lblock, 0)

        pltpu.sync_copy(
            obuf_v,
            out_hbm.at[pl.ds(tt * _TB, _TB), slice(None), pl.ds(w, 1)])

    stage(0, 0)
    fire(0)

    def double_step(tt2, carry):
        tt_e = tt2 * 2
        stage(1, tt_e + 1)
        fire(1)
        drain(0)
        scatter_out(0, tt_e)
        stage(0, tt_e + 2)
        fire(0)
        drain(1)
        scatter_out(1, tt_e + 1)
        return carry

    lax.fori_loop(0, _PAIRS, double_step, 0)

    drain(0)
    scatter_out(0, _NTB - 1)


_CB = 1024
_FULL = _VOCAB // _CB
_TAILC = _VOCAB - _FULL * _CB


@functools.partial(
    pl.kernel,
    out_type=jax.ShapeDtypeStruct((_VOCAB // 4, 128), jnp.float32),
    mesh=_mesh,
    scratch_types=[
        pltpu.VMEM((_EMBED, _CB), jnp.float32),
        pltpu.VMEM((_CB // 4, 128), jnp.float32),
    ],
    compiler_params=pltpu.CompilerParams(use_tc_tiling_on_sc=True,
                                         needs_layout_passes=False),
)
def _table_transpose(tokT_hbm, tailT_hbm, out_hbm, inb_v, outb_v):
    w = lax.axis_index("s") * _NC + lax.axis_index("c")
    nblk = jnp.where(w < 16, 31, 30)
    blk0 = 30 * w + jnp.minimum(w, 16)

    lane = jnp.arange(_L, dtype=jnp.int32)
    zero = jnp.zeros((_L,), dtype=jnp.int32)

    def transpose_cols(ncols):
        def ctile(cb16, carry):
            csplat = jnp.full((_L,), cb16 * _L, dtype=jnp.int32)

            def diag(d, carry2):
                rot = (lane + d) & 15
                cvec = csplat + rot
                v0 = plsc.load_gather(inb_v, [lane, cvec])
                v1 = plsc.load_gather(inb_v, [lane + _L, cvec])
                wo = (cvec << 5) + lane
                plsc.store_scatter(outb_v, [zero, wo], v0)
                plsc.store_scatter(outb_v, [zero, wo + _L], v1)
                return carry2

            lax.fori_loop(0, _L, diag, 0, unroll=4)
            return carry

        lax.fori_loop(0, ncols // _L, ctile, 0)

    def block(i, carry):
        c0 = pl.multiple_of((blk0 + i) * _CB, _CB)
        pltpu.sync_copy(tokT_hbm.at[slice(None), pl.ds(c0, _CB)], inb_v)
        transpose_cols(_CB)
        q0 = pl.multiple_of((blk0 + i) * (_CB // 4), _CB // 4)
        pltpu.sync_copy(outb_v, out_hbm.at[pl.ds(q0, _CB // 4)])
        return carry

    lax.fori_loop(0, nblk, block, 0)

    c0t = pl.multiple_of(_FULL * _CB, 128)
    pltpu.sync_copy(tokT_hbm.at[slice(None), pl.ds(c0t, 512)],
                    inb_v.at[slice(None), pl.ds(0, 512)])
    transpose_cols(512)
    pltpu.sync_copy(outb_v.at[pl.ds(0, 128)],
                    out_hbm.at[pl.ds(_FULL * _CB // 4, 128)])

    pltpu.sync_copy(tailT_hbm, inb_v.at[slice(None), pl.ds(0, 128)])
    transpose_cols(128)
    pltpu.sync_copy(outb_v.at[pl.ds(0, 16)],
                    out_hbm.at[pl.ds(_VOCAB // 4 - 16, 16)])


def kernel(x, token_table, pos_table):
    ntail = _VOCAB - _FULL * _CB - 512
    tail = jnp.pad(token_table[_VOCAB - ntail:].T, ((0, 0), (0, 128 - ntail)))
    table_rm = _table_transpose(
        token_table.T, tail).reshape(_VOCAB, _EMBED)
    out5 = _tok_pos_embed(x.astype(jnp.int32), table_rm, pos_table)
    return out5.transpose(2, 4, 0, 1, 3).reshape(_BATCH, _MAXLEN, _EMBED)


# speedup vs baseline: 6.7640x; 1.1430x over previous
import functools

import jax
import jax.numpy as jnp
from jax import lax
from jax.experimental import pallas as pl
from jax.experimental.pallas import tpu as pltpu
from jax.experimental.pallas import tpu_sc as plsc

_VOCAB = 1_000_000
_MAXLEN = 200
_EMBED = 32
_BATCH = 4096

_L = 16
_NC = 2
_NS = 16
_NW = _NC * _NS
_BB = _BATCH // _NW
_TB = 8
_NTB = _MAXLEN // _TB
_PAIRS = (_NTB - 1) // 2

_mesh = plsc.VectorSubcoreMesh(core_axis_name="c", subcore_axis_name="s")


@functools.partial(
    pl.kernel,
    out_type=jax.ShapeDtypeStruct((_MAXLEN, _EMBED // 8, _NW, 8, 128), jnp.float32),
    mesh=_mesh,
    scratch_types=[
        pltpu.VMEM((_BB, _TB), jnp.int32),
        pltpu.VMEM((_BB, _TB), jnp.int32),
        pltpu.VMEM((_TB, _BB), jnp.int32),
        pltpu.VMEM((_TB, _BB), jnp.int32),
        pltpu.VMEM((_TB * _BB, _EMBED), jnp.float32),
        pltpu.VMEM((_TB * _BB, _EMBED), jnp.float32),
        pltpu.VMEM((_TB, _EMBED // 8, 1, 8, _BB), jnp.float32),
        pltpu.VMEM((_MAXLEN, _EMBED), jnp.float32),
        pltpu.SemaphoreType.DMA,
        pltpu.SemaphoreType.DMA,
    ],
    compiler_params=pltpu.CompilerParams(use_tc_tiling_on_sc=False,
                                         needs_layout_passes=False),
)
def _tok_pos_embed(x_hbm, tok_hbm, pos_hbm, out_hbm,
                   xblk0, xblk1, idx0, idx1, rows0, rows1,
                   obuf_v, pos_v, sem0, sem1):
    w = lax.axis_index("s") * _NC + lax.axis_index("c")
    pltpu.sync_copy(pos_hbm, pos_v)

    lane = jnp.arange(_L, dtype=jnp.int32)
    et_lo = lane >> 3
    es_lo = lane & 7
    et_hi = (lane + _L) >> 3
    es_hi = (lane + _L) & 7
    zero = jnp.zeros((_L,), dtype=jnp.int32)

    xblks = (xblk0, xblk1)
    idxs = (idx0, idx1)
    rows = (rows0, rows1)
    sems = (sem0, sem1)

    def stage(buf, tt):
        xblk, idx_v = xblks[buf], idxs[buf]
        pltpu.sync_copy(
            x_hbm.at[pl.ds(w * _BB, _BB), pl.ds(tt * _TB, _TB)], xblk)
        for s in range(_TB):
            scol = jnp.full((_L,), s, dtype=jnp.int32)
            for lg in range(_BB // _L):
                v = plsc.load_gather(xblk, [lane + (lg * _L), scol])
                idx_v[s, pl.ds(lg * _L, _L)] = v

    def fire(buf):
        for s in range(_TB):
            pltpu.make_async_copy(
                tok_hbm.at[idxs[buf].at[s]],
                rows[buf].at[pl.ds(s * _BB, _BB)],
                sems[buf],
            ).start()

    def drain(buf):
        for s in range(_TB):
            pltpu.make_async_copy(
                tok_hbm.at[idxs[buf].at[s]],
                rows[buf].at[pl.ds(s * _BB, _BB)],
                sems[buf],
            ).wait()

    def scatter_out(buf, tt):
        rows_v = rows[buf]
        lane128 = lane << 7
        for s in range(_TB):
            t = tt * _TB + s
            pos_lo = pos_v[t, pl.ds(0, _L)]
            pos_hi = pos_v[t, pl.ds(_L, _L)]

            def lblock(lb, acc):
                j0 = s * _BB + lb * _L
                jsplat = jnp.full((_L,), j0, dtype=jnp.int32)
                dbase = lane128 + (s * 4096 + lb * _L)

                def diag(d, acc2):
                    rot = (lane + d) & 15
                    jvec = jsplat + rot
                    v0 = plsc.load_gather(rows_v, [jvec, lane]) + pos_lo
                    v1 = plsc.load_gather(rows_v, [jvec, lane + _L]) + pos_hi
                    off = dbase + rot
                    plsc.store_scatter(obuf_v, [zero, zero, zero, zero, off], v0)
                    plsc.store_scatter(obuf_v, [zero, zero, zero, zero, off + 2048], v1)
                    return acc2

                lax.fori_loop(0, _L, diag, 0, unroll=4)
                return acc

            lax.fori_loop(0, _BB // _L, lblock, 0)

        pltpu.sync_copy(
            obuf_v,
            out_hbm.at[pl.ds(tt * _TB, _TB), slice(None), pl.ds(w, 1)])

    stage(0, 0)
    fire(0)

    def double_step(tt2, carry):
        tt_e = tt2 * 2
        stage(1, tt_e + 1)
        fire(1)
        drain(0)
        scatter_out(0, tt_e)
        stage(0, tt_e + 2)
        fire(0)
        drain(1)
        scatter_out(1, tt_e + 1)
        return carry

    lax.fori_loop(0, _PAIRS, double_step, 0)

    drain(0)
    scatter_out(0, _NTB - 1)


_CB = 1024
_FULL = _VOCAB // _CB
_TAILC = _VOCAB - _FULL * _CB


@functools.partial(
    pl.kernel,
    out_type=jax.ShapeDtypeStruct((_VOCAB // 4, 128), jnp.float32),
    mesh=_mesh,
    scratch_types=[
        pltpu.VMEM((_EMBED, _CB), jnp.float32),
        pltpu.VMEM((_EMBED, _CB), jnp.float32),
        pltpu.VMEM((_CB // 4, 128), jnp.float32),
        pltpu.SemaphoreType.DMA,
        pltpu.SemaphoreType.DMA,
    ],
    compiler_params=pltpu.CompilerParams(use_tc_tiling_on_sc=True,
                                         needs_layout_passes=False),
)
def _table_transpose(tokT_hbm, tailT_hbm, out_hbm, inbA, inbB, outb_v,
                     semA, semB):
    w = lax.axis_index("s") * _NC + lax.axis_index("c")
    nblk = jnp.where(w < 16, 31, 30)
    blk0 = 30 * w + jnp.minimum(w, 16)

    lane = jnp.arange(_L, dtype=jnp.int32)
    zero = jnp.zeros((_L,), dtype=jnp.int32)

    inbs = (inbA, inbB)
    sems = (semA, semB)

    def col0(i):
        return pl.multiple_of((blk0 + jnp.minimum(i, nblk - 1)) * _CB, _CB)

    def fire(buf, i):
        pltpu.make_async_copy(
            tokT_hbm.at[slice(None), pl.ds(col0(i), _CB)],
            inbs[buf], sems[buf]).start()

    def drain(buf, i):
        pltpu.make_async_copy(
            tokT_hbm.at[slice(None), pl.ds(col0(i), _CB)],
            inbs[buf], sems[buf]).wait()

    def transpose_cols(inb_v, ncols):
        def ctile(cb16, carry):
            csplat = jnp.full((_L,), cb16 * _L, dtype=jnp.int32)

            def diag(d, carry2):
                rot = (lane + d) & 15
                cvec = csplat + rot
                v0 = plsc.load_gather(inb_v, [lane, cvec])
                v1 = plsc.load_gather(inb_v, [lane + _L, cvec])
                wo = (cvec << 5) + lane
                plsc.store_scatter(outb_v, [zero, wo], v0)
                plsc.store_scatter(outb_v, [zero, wo + _L], v1)
                return carry2

            lax.fori_loop(0, _L, diag, 0, unroll=4)
            return carry

        lax.fori_loop(0, ncols // _L, ctile, 0)

    def emit(buf, i):
        transpose_cols(inbs[buf], _CB)
        q0 = pl.multiple_of(col0(i) // 4, _CB // 4)
        pltpu.sync_copy(outb_v, out_hbm.at[pl.ds(q0, _CB // 4)])

    fire(0, 0)

    def pair(p, carry):
        i_e = p * 2
        fire(1, i_e + 1)
        drain(0, i_e)
        emit(0, i_e)
        fire(0, i_e + 2)
        drain(1, i_e + 1)
        emit(1, i_e + 1)
        return carry

    lax.fori_loop(0, 15, pair, 0)

    drain(0, 30)
    emit(0, 30)

    c0t = pl.multiple_of(_FULL * _CB, 128)
    pltpu.sync_copy(tokT_hbm.at[slice(None), pl.ds(c0t, 512)],
                    inbA.at[slice(None), pl.ds(0, 512)])
    transpose_cols(inbA, 512)
    pltpu.sync_copy(outb_v.at[pl.ds(0, 128)],
                    out_hbm.at[pl.ds(_FULL * _CB // 4, 128)])

    pltpu.sync_copy(tailT_hbm, inbA.at[slice(None), pl.ds(0, 128)])
    transpose_cols(inbA, 128)
    pltpu.sync_copy(outb_v.at[pl.ds(0, 16)],
                    out_hbm.at[pl.ds(_VOCAB // 4 - 16, 16)])


def kernel(x, token_table, pos_table):
    ntail = _VOCAB - _FULL * _CB - 512
    tail = jnp.pad(token_table[_VOCAB - ntail:].T, ((0, 0), (0, 128 - ntail)))
    table_rm = _table_transpose(
        token_table.T, tail).reshape(_VOCAB, _EMBED)
    out5 = _tok_pos_embed(x.astype(jnp.int32), table_rm, pos_table)
    return out5.transpose(2, 4, 0, 1, 3).reshape(_BATCH, _MAXLEN, _EMBED)
